# baseline (speedup 1.0000x reference)
import functools

import jax
import jax.numpy as jnp
from jax import lax
from jax.experimental import pallas as pl
from jax.experimental.pallas import tpu as pltpu
from jax.experimental.pallas import tpu_sc as plsc

DIM = 64
L = 16
SUB = 64
PAIR = 2 * DIM


def _sc_partial_scores(ent2_re, ent2_im, rel2_re, rel2_im, h_idx, r_idx, t_idx):
    b_total = h_idx.shape[0]
    info = plsc.get_sparse_core_info()
    nw = info.num_cores * info.num_subcores
    chunk = b_total // nw
    nsub = chunk // SUB
    assert chunk % SUB == 0 and SUB % L == 0

    mesh = plsc.VectorSubcoreMesh(core_axis_name="c", subcore_axis_name="s")

    @functools.partial(
        pl.kernel,
        mesh=mesh,
        out_type=jax.ShapeDtypeStruct((b_total * L,), jnp.float32),
        scratch_types=[
            pltpu.VMEM((chunk,), jnp.int32),
            pltpu.VMEM((chunk,), jnp.int32),
            pltpu.VMEM((chunk,), jnp.int32),
            pltpu.VMEM((chunk,), jnp.int32),
            pltpu.VMEM((chunk,), jnp.int32),
            pltpu.VMEM((chunk,), jnp.int32),
            pltpu.VMEM((SUB, PAIR), jnp.float32),
            pltpu.VMEM((SUB, PAIR), jnp.float32),
            pltpu.VMEM((SUB, PAIR), jnp.float32),
            pltpu.VMEM((SUB, PAIR), jnp.float32),
            pltpu.VMEM((SUB, PAIR), jnp.float32),
            pltpu.VMEM((SUB, PAIR), jnp.float32),
            pltpu.VMEM((chunk * L,), jnp.float32),
            pltpu.SemaphoreType.DMA,
        ],
    )
    def sc_kernel(ent_re_hbm, ent_im_hbm, rel_re_hbm, rel_im_hbm,
                  h_hbm, r_hbm, t_hbm, out_hbm,
                  h2_v, r2_v, t2_v, ho_v, ro_v, to_v,
                  hre_v, him_v, tre_v, tim_v, rre_v, rim_v,
                  part_v, sem):
        wid = lax.axis_index("s") * info.num_cores + lax.axis_index("c")
        base = wid * chunk
        pltpu.sync_copy(h_hbm.at[pl.ds(base, chunk)], h2_v)
        pltpu.sync_copy(r_hbm.at[pl.ds(base, chunk)], r2_v)
        pltpu.sync_copy(t_hbm.at[pl.ds(base, chunk)], t2_v)

        def split_body(j, carry):
            sl = pl.ds(j * L, L)
            hv = h2_v[sl]
            rv = r2_v[sl]
            tv = t2_v[sl]
            ho_v[sl] = (hv & 1) * DIM
            ro_v[sl] = (rv & 1) * DIM
            to_v[sl] = (tv & 1) * DIM
            h2_v[sl] = lax.shift_right_logical(hv, 1)
            r2_v[sl] = lax.shift_right_logical(rv, 1)
            t2_v[sl] = lax.shift_right_logical(tv, 1)
            return carry

        lax.fori_loop(0, chunk // L, split_body, 0)

        def sub_body(s, carry):
            off = s * SUB
            cps = [
                pltpu.async_copy(ent_re_hbm.at[h2_v.at[pl.ds(off, SUB)]], hre_v, sem),
                pltpu.async_copy(ent_im_hbm.at[h2_v.at[pl.ds(off, SUB)]], him_v, sem),
                pltpu.async_copy(ent_re_hbm.at[t2_v.at[pl.ds(off, SUB)]], tre_v, sem),
                pltpu.async_copy(ent_im_hbm.at[t2_v.at[pl.ds(off, SUB)]], tim_v, sem),
                pltpu.async_copy(rel_re_hbm.at[r2_v.at[pl.ds(off, SUB)]], rre_v, sem),
                pltpu.async_copy(rel_im_hbm.at[r2_v.at[pl.ds(off, SUB)]], rim_v, sem),
            ]
            for cp in cps:
                cp.wait()

            def wave_body(wv, c2):
                row0 = wv * L
                ho16 = ho_v[pl.ds(off + row0, L)]
                ro16 = ro_v[pl.ds(off + row0, L)]
                to16 = to_v[pl.ds(off + row0, L)]
                for g in range(L):
                    i = row0 + g
                    oh = ho16[g]
                    orr = ro16[g]
                    ot = to16[g]
                    acc = jnp.zeros((L,), jnp.float32)
                    for c in range(DIM // L):
                        hre = hre_v[i, pl.ds(oh + c * L, L)]
                        him = him_v[i, pl.ds(oh + c * L, L)]
                        tre = tre_v[i, pl.ds(ot + c * L, L)]
                        tim = tim_v[i, pl.ds(ot + c * L, L)]
                        rre = rre_v[i, pl.ds(orr + c * L, L)]
                        rim = rim_v[i, pl.ds(orr + c * L, L)]
                        acc = acc + rre * (hre * tre + him * tim) + rim * (hre * tim - him * tre)
                    part_v[pl.ds((off + i) * L, L)] = acc
                return c2

            lax.fori_loop(0, SUB // L, wave_body, 0)
            return carry

        lax.fori_loop(0, nsub, sub_body, 0)
        pltpu.sync_copy(part_v, out_hbm.at[pl.ds(base * L, chunk * L)])

    return sc_kernel(ent2_re, ent2_im, rel2_re, rel2_im, h_idx, r_idx, t_idx)


def _loss_tc_kernel(part_ref, out_ref):
    x = part_ref[...]
    s = jnp.sum(x, axis=2)
    sgn = jnp.concatenate(
        [jnp.full((1, s.shape[1]), -1.0, jnp.float32),
         jnp.full((1, s.shape[1]), 1.0, jnp.float32)], axis=0)
    z = s * sgn
    sp = jnp.maximum(z, 0.0) + jnp.log1p(jnp.exp(-jnp.abs(z)))
    out_ref[...] = jnp.mean(sp, axis=(0, 1), keepdims=True).reshape(1, 1)


def kernel(ent_re, ent_im, rel_re, rel_im, positive_triples, negative_triples):
    b = positive_triples.shape[0]
    h_idx = jnp.concatenate(
        [positive_triples[:, 0], negative_triples[:, 0]]).astype(jnp.int32)
    r_idx = jnp.concatenate(
        [positive_triples[:, 1], negative_triples[:, 1]]).astype(jnp.int32)
    t_idx = jnp.concatenate(
        [positive_triples[:, 2], negative_triples[:, 2]]).astype(jnp.int32)

    ent2_re = ent_re.reshape(-1, PAIR)
    ent2_im = ent_im.reshape(-1, PAIR)
    rel2_re = rel_re.reshape(-1, PAIR)
    rel2_im = rel_im.reshape(-1, PAIR)

    part = _sc_partial_scores(ent2_re, ent2_im, rel2_re, rel2_im,
                              h_idx, r_idx, t_idx)
    part3 = part.reshape(2, b, L)

    loss = pl.pallas_call(
        _loss_tc_kernel,
        out_shape=jax.ShapeDtypeStruct((1, 1), jnp.float32),
    )(part3)
    return loss.reshape(())


# speedup vs baseline: 1.3239x; 1.3239x over previous
import functools

import jax
import jax.numpy as jnp
from jax import lax
from jax.experimental import pallas as pl
from jax.experimental.pallas import tpu as pltpu
from jax.experimental.pallas import tpu_sc as plsc

DIM = 64
L = 16
GRP = 8
WAVE = 16


def _sc_partial_scores(ent3_re, ent3_im, rel3_re, rel3_im, h_idx, r_idx, t_idx):
    b_total = h_idx.shape[0]
    nrel_grp = rel3_re.shape[0]
    info = plsc.get_sparse_core_info()
    nw = info.num_cores * info.num_subcores
    chunk = b_total // nw
    nwave = chunk // WAVE
    assert chunk % WAVE == 0

    mesh = plsc.VectorSubcoreMesh(core_axis_name="c", subcore_axis_name="s")

    @functools.partial(
        pl.kernel,
        mesh=mesh,
        out_type=jax.ShapeDtypeStruct((b_total * L,), jnp.float32),
        scratch_types=[
            pltpu.VMEM((chunk,), jnp.int32),
            pltpu.VMEM((chunk,), jnp.int32),
            pltpu.VMEM((chunk,), jnp.int32),
            pltpu.VMEM((WAVE, GRP, DIM), jnp.float32),
            pltpu.VMEM((WAVE, GRP, DIM), jnp.float32),
            pltpu.VMEM((WAVE, GRP, DIM), jnp.float32),
            pltpu.VMEM((WAVE, GRP, DIM), jnp.float32),
            pltpu.VMEM((WAVE, DIM), jnp.float32),
            pltpu.VMEM((WAVE, DIM), jnp.float32),
            pltpu.VMEM((chunk * L,), jnp.float32),
            pltpu.VMEM_SHARED((nrel_grp, GRP, DIM), jnp.float32),
            pltpu.VMEM_SHARED((nrel_grp, GRP, DIM), jnp.float32),
            pltpu.SemaphoreType.DMA,
        ],
    )
    def sc_kernel(ent_re_hbm, ent_im_hbm, rel_re_hbm, rel_im_hbm,
                  h_hbm, r_hbm, t_hbm, out_hbm,
                  h_v, r_v, t_v, hre_v, him_v, tre_v, tim_v, rre_v, rim_v,
                  part_v, relre_sp, relim_sp, sem):
        cid = lax.axis_index("c")
        sid = lax.axis_index("s")
        wid = sid * info.num_cores + cid
        base = wid * chunk

        @pl.when(sid == 0)
        def _stage_rel():
            def stage_body(j, carry):
                pltpu.sync_copy(rel_re_hbm.at[j], relre_sp.at[j])
                pltpu.sync_copy(rel_im_hbm.at[j], relim_sp.at[j])
                return carry
            lax.fori_loop(0, nrel_grp, stage_body, 0)

        pltpu.sync_copy(h_hbm.at[pl.ds(base, chunk)], h_v)
        pltpu.sync_copy(r_hbm.at[pl.ds(base, chunk)], r_v)
        pltpu.sync_copy(t_hbm.at[pl.ds(base, chunk)], t_v)
        plsc.subcore_barrier()

        def wave_body(w, carry):
            row0 = w * WAVE
            hv16 = h_v[pl.ds(row0, L)]
            tv16 = t_v[pl.ds(row0, L)]
            rv16 = r_v[pl.ds(row0, L)]
            cps = []
            for g in range(WAVE):
                ih = hv16[g]
                it = tv16[g]
                ir = rv16[g]
                cps.append(pltpu.async_copy(
                    ent_re_hbm.at[lax.shift_right_logical(ih, 3)], hre_v.at[g], sem))
                cps.append(pltpu.async_copy(
                    ent_im_hbm.at[lax.shift_right_logical(ih, 3)], him_v.at[g], sem))
                cps.append(pltpu.async_copy(
                    ent_re_hbm.at[lax.shift_right_logical(it, 3)], tre_v.at[g], sem))
                cps.append(pltpu.async_copy(
                    ent_im_hbm.at[lax.shift_right_logical(it, 3)], tim_v.at[g], sem))
                cps.append(pltpu.async_copy(
                    relre_sp.at[lax.shift_right_logical(ir, 3), ir & 7], rre_v.at[g], sem))
                cps.append(pltpu.async_copy(
                    relim_sp.at[lax.shift_right_logical(ir, 3), ir & 7], rim_v.at[g], sem))
            for cp in cps:
                cp.wait()
            for g in range(WAVE):
                rh = hv16[g] & 7
                rt = tv16[g] & 7
                acc = jnp.zeros((L,), jnp.float32)
                for c in range(DIM // L):
                    sl = pl.ds(c * L, L)
                    hre = hre_v[g, rh, sl]
                    him = him_v[g, rh, sl]
                    tre = tre_v[g, rt, sl]
                    tim = tim_v[g, rt, sl]
                    rre = rre_v[g, sl]
                    rim = rim_v[g, sl]
                    acc = acc + rre * (hre * tre + him * tim) + rim * (hre * tim - him * tre)
                part_v[pl.ds((row0 + g) * L, L)] = acc
            return carry

        lax.fori_loop(0, nwave, wave_body, 0)
        pltpu.sync_copy(part_v, out_hbm.at[pl.ds(base * L, chunk * L)])

    return sc_kernel(ent3_re, ent3_im, rel3_re, rel3_im, h_idx, r_idx, t_idx)


def _loss_tc_kernel(part_ref, out_ref):
    x = part_ref[...]
    s = jnp.sum(x, axis=2)
    sgn = jnp.concatenate(
        [jnp.full((1, s.shape[1]), -1.0, jnp.float32),
         jnp.full((1, s.shape[1]), 1.0, jnp.float32)], axis=0)
    z = s * sgn
    sp = jnp.maximum(z, 0.0) + jnp.log1p(jnp.exp(-jnp.abs(z)))
    out_ref[...] = jnp.mean(sp, axis=(0, 1), keepdims=True).reshape(1, 1)


def kernel(ent_re, ent_im, rel_re, rel_im, positive_triples, negative_triples):
    b = positive_triples.shape[0]
    h_idx = jnp.concatenate(
        [positive_triples[:, 0], negative_triples[:, 0]]).astype(jnp.int32)
    r_idx = jnp.concatenate(
        [positive_triples[:, 1], negative_triples[:, 1]]).astype(jnp.int32)
    t_idx = jnp.concatenate(
        [positive_triples[:, 2], negative_triples[:, 2]]).astype(jnp.int32)

    ent3_re = ent_re.reshape(-1, GRP, DIM)
    ent3_im = ent_im.reshape(-1, GRP, DIM)
    rel3_re = rel_re.reshape(-1, GRP, DIM)
    rel3_im = rel_im.reshape(-1, GRP, DIM)

    part = _sc_partial_scores(ent3_re, ent3_im, rel3_re, rel3_im,
                              h_idx, r_idx, t_idx)
    part3 = part.reshape(2, b, L)

    loss = pl.pallas_call(
        _loss_tc_kernel,
        out_shape=jax.ShapeDtypeStruct((1, 1), jnp.float32),
    )(part3)
    return loss.reshape(())
